# baseline (speedup 1.0000x reference)
import math

import jax
import jax.numpy as jnp
import numpy as np
from jax.experimental import pallas as pl
from jax.experimental.pallas import tpu as pltpu

B = 1024
DIM = 512
C = 1211
K = 3
CN = C * K
R = 0.4
WL = 0.3
TOPK = math.ceil(R * CN)

W = 3712
Cp = 1280
RB = 128
CT = 128

_INT_MIN = np.uint32(0x80000000).view(np.int32)

_CLS_ALL = (np.arange(W, dtype=np.int32) // 3).reshape(1, W)
_CLS_START = np.where(np.arange(W) % 3 == 0, _CLS_ALL[0], 4096).reshape(1, W)
_CLS_START = _CLS_START.astype(np.int32)


def _shift_left(m, n):
    return jnp.concatenate([m[:, n:], jnp.zeros((m.shape[0], n), m.dtype)],
                           axis=1)


def _prep_kernel(p_ref, inv_ref, zm_ref):
    p = p_ref[...]
    ssq = jnp.sum(p * p, axis=0, keepdims=True)
    inv = 1.0 / jnp.maximum(jnp.sqrt(ssq), 1e-12)
    inv_ref[...] = inv
    t = p * inv
    zm_ref[...] = t + _shift_left(t, 1) + _shift_left(t, 2)


def _loss_kernel(x_ref, p_ref, inv_ref, tgt_ref, ca_ref, cs_ref, out_ref):
    x = x_ref[...]
    p = p_ref[...]
    s = jax.lax.dot_general(x, p, (((1,), (0,)), ((), ())),
                            preferred_element_type=jnp.float32)
    s = s * inv_ref[...]

    cls_all = ca_ref[...]
    cls_start = cs_ref[...]
    tgt = tgt_ref[:, 0:1]

    valid = cls_all < C
    boosted = jnp.where(valid,
                        s + jnp.where(cls_all == tgt, 100.0, 0.0),
                        -3e38)
    u = jax.lax.bitcast_convert_type(boosted, jnp.int32)
    es = jnp.where(u >= 0, u, u ^ np.int32(0x7FFFFFFF))

    topk16 = np.int16(TOPK)
    bias16 = np.uint16(0x8000).view(np.int16)

    def count16(mask):
        ones = jnp.where(mask, jnp.bfloat16(1), jnp.bfloat16(0))
        acc = ones[:, :128]
        for i in range(1, W // 128):
            acc = acc + ones[:, i * 128:(i + 1) * 128]
        cnt = jnp.sum(acc.astype(jnp.float32), axis=1, keepdims=True)
        return cnt.astype(jnp.int32).astype(jnp.int16)

    hi = (es >> 16).astype(jnp.int16)
    t_hi_u = jnp.zeros((RB, 1), jnp.int16)
    for b in range(15, -1, -1):
        cand_u = t_hi_u | np.uint16(1 << b).view(np.int16)
        cand_s = cand_u ^ bias16
        cnt = count16(hi >= cand_s)
        t_hi_u = jnp.where(cnt >= topk16, cand_u, t_hi_u)
    t_hi = t_hi_u ^ bias16

    c_eq = count16(hi == t_hi)
    c_ge = count16(hi >= t_hi)
    kk2 = topk16 - (c_ge - c_eq)

    lo16 = ((es & np.int32(0xFFFF)) - 32768).astype(jnp.int16)
    act = jnp.where(hi == t_hi, lo16, np.int16(-32768))
    t_lo_u = jnp.zeros((RB, 1), jnp.int16)
    for b in range(15, -1, -1):
        cand_u = t_lo_u | np.uint16(1 << b).view(np.int16)
        cand_s = cand_u ^ bias16
        cnt = count16(act >= cand_s)
        t_lo_u = jnp.where(cnt >= kk2, cand_u, t_lo_u)

    t_s = ((t_hi.astype(jnp.int32)) << 16) | (t_lo_u.astype(jnp.int32) & 0xFFFF)

    sel = es >= t_s
    m = jnp.where(sel, s, 0.0)
    lg = m + _shift_left(m, 1) + _shift_left(m, 2)

    live = (cls_start < C) & (lg != 0.0)
    se = jnp.where(live, jnp.exp(lg), 0.0)
    denom = 1e-8 + jnp.sum(se, axis=1, keepdims=True)
    texp = jnp.sum(jnp.where(cls_start == tgt, se, 0.0), axis=1, keepdims=True)
    lossrow = -jnp.log(texp / denom + 1e-20)
    partial = jnp.sum(lossrow, keepdims=True).reshape(1, 1) * (1.0 / B)

    @pl.when(pl.program_id(0) == 0)
    def _():
        out_ref[...] = jnp.zeros((1, 1), jnp.float32)

    out_ref[...] += partial


def _reg_kernel(p_ref, invc_ref, cls_ref, z_ref, out_ref):
    pb = p_ref[...]
    invc = invc_ref[...]
    z = z_ref[...]
    cl = jax.lax.dot_general(pb, z, (((0,), (0,)), ((), ())),
                             preferred_element_type=jnp.float32)
    cl = cl * invc

    ci = cls_ref[...]
    rvalid = ci < C

    colv = jax.lax.broadcasted_iota(jnp.int32, (CT, Cp), 1)
    clm = jnp.where(colv < C, cl, -3e38)
    mx = jnp.max(clm, axis=1, keepdims=True)
    lse = mx + jnp.log(jnp.sum(jnp.exp(clm - mx), axis=1, keepdims=True))
    diag = jnp.sum(jnp.where(colv == ci, cl, 0.0), axis=1, keepdims=True)
    contrib = jnp.where(rvalid, lse - diag, 0.0)

    @pl.when(pl.program_id(0) == 0)
    def _():
        out_ref[...] = jnp.zeros((1, 1), jnp.float32)

    out_ref[...] += jnp.sum(contrib, keepdims=True).reshape(1, 1) * (1.0 / CN)


@jax.jit
def kernel(input, proxies, target):
    pw = jnp.pad(proxies, ((0, 0), (0, W - CN)))
    tgt2d = jnp.broadcast_to(target[:, None], (B, RB)).astype(jnp.int32)

    inv, zm = pl.pallas_call(
        _prep_kernel,
        out_shape=(
            jax.ShapeDtypeStruct((1, W), jnp.float32),
            jax.ShapeDtypeStruct((DIM, W), jnp.float32),
        ),
    )(pw)

    z = jnp.pad(zm[:, 0:CN:3], ((0, 0), (0, Cp - C)))
    inv_col = jnp.reshape(inv, (W, 1))
    cls_col = jnp.asarray(_CLS_ALL.reshape(W, 1))

    loss_cls = pl.pallas_call(
        _loss_kernel,
        grid=(B // RB,),
        in_specs=[
            pl.BlockSpec((RB, DIM), lambda i: (i, 0)),
            pl.BlockSpec((DIM, W), lambda i: (0, 0)),
            pl.BlockSpec((1, W), lambda i: (0, 0)),
            pl.BlockSpec((RB, RB), lambda i: (i, 0)),
            pl.BlockSpec((1, W), lambda i: (0, 0)),
            pl.BlockSpec((1, W), lambda i: (0, 0)),
        ],
        out_specs=pl.BlockSpec((1, 1), lambda i: (0, 0)),
        out_shape=jax.ShapeDtypeStruct((1, 1), jnp.float32),
    )(input, pw, inv, tgt2d, jnp.asarray(_CLS_ALL), jnp.asarray(_CLS_START))

    reg = pl.pallas_call(
        _reg_kernel,
        grid=(W // CT,),
        in_specs=[
            pl.BlockSpec((DIM, CT), lambda i: (0, i)),
            pl.BlockSpec((CT, 1), lambda i: (i, 0)),
            pl.BlockSpec((CT, 1), lambda i: (i, 0)),
            pl.BlockSpec((DIM, Cp), lambda i: (0, 0)),
        ],
        out_specs=pl.BlockSpec((1, 1), lambda i: (0, 0)),
        out_shape=jax.ShapeDtypeStruct((1, 1), jnp.float32),
    )(pw, inv_col, cls_col, z)

    return loss_cls[0, 0] + WL * reg[0, 0]


# speedup vs baseline: 1.4320x; 1.4320x over previous
import math

import jax
import jax.numpy as jnp
import numpy as np
from jax.experimental import pallas as pl
from jax.experimental.pallas import tpu as pltpu

B = 1024
DIM = 512
C = 1211
K = 3
CN = C * K
R = 0.4
WL = 0.3
TOPK = math.ceil(R * CN)

Cp = 1280
W = K * Cp
RB = 128
CT = 256

_CLS_PLANE = (np.arange(W, dtype=np.int32) % Cp).reshape(1, W)
_CLS_CP = np.arange(Cp, dtype=np.int32).reshape(1, Cp)


def _loss_kernel(x_ref, p_ref, tgt_ref, ca_ref, cc_ref, out_ref, z_ref,
                 inv_ref):
    p = p_ref[...]

    @pl.when(pl.program_id(0) == 0)
    def _():
        ssq = jnp.sum(p * p, axis=0, keepdims=True)
        inv = 1.0 / jnp.maximum(jnp.sqrt(ssq), 1e-12)
        inv_ref[...] = inv
        t = p * inv
        z_ref[...] = t[:, :Cp] + t[:, Cp:2 * Cp] + t[:, 2 * Cp:]
        out_ref[...] = jnp.zeros((1, 1), jnp.float32)

    x = x_ref[...]
    s = jax.lax.dot_general(x, p, (((1,), (0,)), ((), ())),
                            preferred_element_type=jnp.float32)
    s = s * inv_ref[...]

    cls_all = ca_ref[...]
    cls_cp = cc_ref[...]
    tgt = tgt_ref[:, 0:1]

    valid = cls_all < C
    boosted = jnp.where(valid,
                        s + jnp.where(cls_all == tgt, 100.0, 0.0),
                        -3e38)
    u = jax.lax.bitcast_convert_type(boosted, jnp.int32)
    es = jnp.where(u >= 0, u, u ^ np.int32(0x7FFFFFFF))

    topk16 = np.int16(TOPK)
    bias16 = np.uint16(0x8000).view(np.int16)

    def count16(mask):
        ones = jnp.where(mask, jnp.bfloat16(1), jnp.bfloat16(0))
        acc = ones[:, :128]
        for i in range(1, W // 128):
            acc = acc + ones[:, i * 128:(i + 1) * 128]
        cnt = jnp.sum(acc.astype(jnp.float32), axis=1, keepdims=True)
        return cnt.astype(jnp.int32).astype(jnp.int16)

    hi = (es >> 16).astype(jnp.int16)
    t_hi_u = jnp.zeros((RB, 1), jnp.int16)
    for b in range(15, -1, -1):
        cand_u = t_hi_u | np.uint16(1 << b).view(np.int16)
        cand_s = cand_u ^ bias16
        cnt = count16(hi >= cand_s)
        t_hi_u = jnp.where(cnt >= topk16, cand_u, t_hi_u)
    t_hi = t_hi_u ^ bias16

    c_eq = count16(hi == t_hi)
    c_ge = count16(hi >= t_hi)
    kk2 = topk16 - (c_ge - c_eq)

    lo16 = ((es & np.int32(0xFFFF)) - 32768).astype(jnp.int16)
    act = jnp.where(hi == t_hi, lo16, np.int16(-32768))
    t_lo_u = jnp.zeros((RB, 1), jnp.int16)
    for b in range(15, -1, -1):
        cand_u = t_lo_u | np.uint16(1 << b).view(np.int16)
        cand_s = cand_u ^ bias16
        cnt = count16(act >= cand_s)
        t_lo_u = jnp.where(cnt >= kk2, cand_u, t_lo_u)

    t_s = ((t_hi.astype(jnp.int32)) << 16) | (t_lo_u.astype(jnp.int32) & 0xFFFF)

    sel = es >= t_s
    m = jnp.where(sel, s, 0.0)
    logits = m[:, :Cp] + m[:, Cp:2 * Cp] + m[:, 2 * Cp:]

    se = jnp.where(logits != 0.0, jnp.exp(logits), 0.0)
    denom = 1e-8 + jnp.sum(se, axis=1, keepdims=True)
    texp = jnp.sum(jnp.where(cls_cp == tgt, se, 0.0), axis=1, keepdims=True)
    lossrow = -jnp.log(texp / denom + 1e-20)
    out_ref[...] += jnp.sum(lossrow, keepdims=True).reshape(1, 1) * (1.0 / B)


def _reg_kernel(p_ref, cls_ref, z_ref, out_ref):
    pb = p_ref[...]
    z = z_ref[...]
    ssq = jnp.sum(pb * pb, axis=0, keepdims=True)
    inv = 1.0 / jnp.maximum(jnp.sqrt(ssq), 1e-12)
    clt = jax.lax.dot_general(z, pb, (((0,), (0,)), ((), ())),
                              preferred_element_type=jnp.float32)
    clt = clt * inv

    ci = cls_ref[...]
    cvalid = ci < C

    rows = jax.lax.broadcasted_iota(jnp.int32, (Cp, CT), 0)
    clm = jnp.where(rows < C, clt, -3e38)
    mx = jnp.max(clm, axis=0, keepdims=True)
    lse = mx + jnp.log(jnp.sum(jnp.exp(clm - mx), axis=0, keepdims=True))
    diag = jnp.sum(jnp.where(rows == ci, clt, 0.0), axis=0, keepdims=True)
    contrib = jnp.where(cvalid, lse - diag, 0.0)

    @pl.when(pl.program_id(0) == 0)
    def _():
        out_ref[...] = jnp.zeros((1, 1), jnp.float32)

    out_ref[...] += jnp.sum(contrib, keepdims=True).reshape(1, 1) * (1.0 / CN)


@jax.jit
def kernel(input, proxies, target):
    pr = proxies.reshape(DIM, C, K)
    planes = [jnp.pad(pr[:, :, k], ((0, 0), (0, Cp - C))) for k in range(K)]
    pcat = jnp.concatenate(planes, axis=1)
    tgt2d = jnp.broadcast_to(target[:, None], (B, RB)).astype(jnp.int32)

    loss_cls, z, _ = pl.pallas_call(
        _loss_kernel,
        grid=(B // RB,),
        in_specs=[
            pl.BlockSpec((RB, DIM), lambda i: (i, 0)),
            pl.BlockSpec((DIM, W), lambda i: (0, 0)),
            pl.BlockSpec((RB, RB), lambda i: (i, 0)),
            pl.BlockSpec((1, W), lambda i: (0, 0)),
            pl.BlockSpec((1, Cp), lambda i: (0, 0)),
        ],
        out_specs=(
            pl.BlockSpec((1, 1), lambda i: (0, 0)),
            pl.BlockSpec((DIM, Cp), lambda i: (0, 0)),
            pl.BlockSpec((1, W), lambda i: (0, 0)),
        ),
        out_shape=(
            jax.ShapeDtypeStruct((1, 1), jnp.float32),
            jax.ShapeDtypeStruct((DIM, Cp), jnp.float32),
            jax.ShapeDtypeStruct((1, W), jnp.float32),
        ),
    )(input, pcat, tgt2d, jnp.asarray(_CLS_PLANE), jnp.asarray(_CLS_CP))

    reg = pl.pallas_call(
        _reg_kernel,
        grid=(W // CT,),
        in_specs=[
            pl.BlockSpec((DIM, CT), lambda i: (0, i)),
            pl.BlockSpec((1, CT), lambda i: (0, i)),
            pl.BlockSpec((DIM, Cp), lambda i: (0, 0)),
        ],
        out_specs=pl.BlockSpec((1, 1), lambda i: (0, 0)),
        out_shape=jax.ShapeDtypeStruct((1, 1), jnp.float32),
    )(pcat, jnp.asarray(_CLS_PLANE), z)

    return loss_cls[0, 0] + WL * reg[0, 0]


# speedup vs baseline: 1.5533x; 1.0848x over previous
import math

import jax
import jax.numpy as jnp
import numpy as np
from jax.experimental import pallas as pl
from jax.experimental.pallas import tpu as pltpu

B = 1024
DIM = 512
C = 1211
K = 3
CN = C * K
R = 0.4
WL = 0.3
TOPK = math.ceil(R * CN)

Cp = 1280
W = K * Cp
RB = 128
CT = 256

_CLS_PLANE = (np.arange(W, dtype=np.int32) % Cp).reshape(1, W)
_CLS_CP = np.arange(Cp, dtype=np.int32).reshape(1, Cp)


def _loss_kernel(x_ref, p_ref, tgt_ref, ca_ref, cc_ref, out_ref, z_ref,
                 inv_ref):
    p = p_ref[...]

    @pl.when(pl.program_id(0) == 0)
    def _():
        pf = p.astype(jnp.float32)
        ssq = jnp.sum(pf * pf, axis=0, keepdims=True)
        inv = 1.0 / jnp.maximum(jnp.sqrt(ssq), 1e-12)
        inv_ref[...] = inv
        t = pf * inv
        z_ref[...] = t[:, :Cp] + t[:, Cp:2 * Cp] + t[:, 2 * Cp:]
        out_ref[...] = jnp.zeros((1, 1), jnp.float32)

    x = x_ref[...]
    s = jax.lax.dot_general(x, p, (((1,), (0,)), ((), ())),
                            preferred_element_type=jnp.float32)
    s = s * inv_ref[...]

    cls_all = ca_ref[...]
    cls_cp = cc_ref[...]
    tgt = tgt_ref[:, 0:1]

    valid = cls_all < C
    boosted = jnp.where(valid,
                        s + jnp.where(cls_all == tgt, 100.0, 0.0),
                        -3e38)
    u = jax.lax.bitcast_convert_type(boosted, jnp.int32)
    es = jnp.where(u >= 0, u, u ^ np.int32(0x7FFFFFFF))

    topk16 = np.int16(TOPK)
    bias16 = np.uint16(0x8000).view(np.int16)

    def count16(mask):
        ones = jnp.where(mask, jnp.bfloat16(1), jnp.bfloat16(0))
        acc = ones[:, :128]
        for i in range(1, W // 128):
            acc = acc + ones[:, i * 128:(i + 1) * 128]
        cnt = jnp.sum(acc.astype(jnp.float32), axis=1, keepdims=True)
        return cnt.astype(jnp.int32).astype(jnp.int16)

    hi = (es >> 16).astype(jnp.int16)
    t_hi_u = jnp.zeros((RB, 1), jnp.int16)
    for b in range(15, -1, -1):
        cand_u = t_hi_u | np.uint16(1 << b).view(np.int16)
        cand_s = cand_u ^ bias16
        cnt = count16(hi >= cand_s)
        t_hi_u = jnp.where(cnt >= topk16, cand_u, t_hi_u)
    t_hi = t_hi_u ^ bias16

    c_eq = count16(hi == t_hi)
    c_ge = count16(hi >= t_hi)
    kk2 = topk16 - (c_ge - c_eq)

    lo16 = ((es & np.int32(0xFFFF)) - 32768).astype(jnp.int16)
    act = jnp.where(hi == t_hi, lo16, np.int16(-32768))
    t_lo_u = jnp.zeros((RB, 1), jnp.int16)
    for b in range(15, -1, -1):
        cand_u = t_lo_u | np.uint16(1 << b).view(np.int16)
        cand_s = cand_u ^ bias16
        cnt = count16(act >= cand_s)
        t_lo_u = jnp.where(cnt >= kk2, cand_u, t_lo_u)

    t_s = ((t_hi.astype(jnp.int32)) << 16) | (t_lo_u.astype(jnp.int32) & 0xFFFF)

    sel = es >= t_s
    m = jnp.where(sel, s, 0.0)
    logits = m[:, :Cp] + m[:, Cp:2 * Cp] + m[:, 2 * Cp:]

    se = jnp.where(logits != 0.0, jnp.exp(logits), 0.0)
    denom = 1e-8 + jnp.sum(se, axis=1, keepdims=True)
    texp = jnp.sum(jnp.where(cls_cp == tgt, se, 0.0), axis=1, keepdims=True)
    lossrow = -jnp.log(texp / denom + 1e-20)
    out_ref[...] += jnp.sum(lossrow, keepdims=True).reshape(1, 1) * (1.0 / B)


def _reg_kernel(p_ref, cls_ref, z_ref, out_ref):
    pb = p_ref[...]
    z = z_ref[...]
    pbf = pb.astype(jnp.float32)
    ssq = jnp.sum(pbf * pbf, axis=0, keepdims=True)
    inv = 1.0 / jnp.maximum(jnp.sqrt(ssq), 1e-12)
    clt = jax.lax.dot_general(z, pb, (((0,), (0,)), ((), ())),
                              preferred_element_type=jnp.float32)
    clt = clt * inv

    ci = cls_ref[...]
    cvalid = ci < C

    rows = jax.lax.broadcasted_iota(jnp.int32, (Cp, CT), 0)
    clm = jnp.where(rows < C, clt, -3e38)
    mx = jnp.max(clm, axis=0, keepdims=True)
    lse = mx + jnp.log(jnp.sum(jnp.exp(clm - mx), axis=0, keepdims=True))
    diag = jnp.sum(jnp.where(rows == ci, clt, 0.0), axis=0, keepdims=True)
    contrib = jnp.where(cvalid, lse - diag, 0.0)

    @pl.when(pl.program_id(0) == 0)
    def _():
        out_ref[...] = jnp.zeros((1, 1), jnp.float32)

    out_ref[...] += jnp.sum(contrib, keepdims=True).reshape(1, 1) * (1.0 / CN)


@jax.jit
def kernel(input, proxies, target):
    pr = proxies.reshape(DIM, C, K)
    planes = [jnp.pad(pr[:, :, k], ((0, 0), (0, Cp - C))) for k in range(K)]
    pcat = jnp.concatenate(planes, axis=1).astype(jnp.bfloat16)
    x16 = input.astype(jnp.bfloat16)
    tgt2d = jnp.broadcast_to(target[:, None], (B, RB)).astype(jnp.int32)

    loss_cls, z, _ = pl.pallas_call(
        _loss_kernel,
        grid=(B // RB,),
        in_specs=[
            pl.BlockSpec((RB, DIM), lambda i: (i, 0)),
            pl.BlockSpec((DIM, W), lambda i: (0, 0)),
            pl.BlockSpec((RB, RB), lambda i: (i, 0)),
            pl.BlockSpec((1, W), lambda i: (0, 0)),
            pl.BlockSpec((1, Cp), lambda i: (0, 0)),
        ],
        out_specs=(
            pl.BlockSpec((1, 1), lambda i: (0, 0)),
            pl.BlockSpec((DIM, Cp), lambda i: (0, 0)),
            pl.BlockSpec((1, W), lambda i: (0, 0)),
        ),
        out_shape=(
            jax.ShapeDtypeStruct((1, 1), jnp.float32),
            jax.ShapeDtypeStruct((DIM, Cp), jnp.float32),
            jax.ShapeDtypeStruct((1, W), jnp.float32),
        ),
    )(x16, pcat, tgt2d, jnp.asarray(_CLS_PLANE), jnp.asarray(_CLS_CP))

    reg = pl.pallas_call(
        _reg_kernel,
        grid=(W // CT,),
        in_specs=[
            pl.BlockSpec((DIM, CT), lambda i: (0, i)),
            pl.BlockSpec((1, CT), lambda i: (0, i)),
            pl.BlockSpec((DIM, Cp), lambda i: (0, 0)),
        ],
        out_specs=pl.BlockSpec((1, 1), lambda i: (0, 0)),
        out_shape=jax.ShapeDtypeStruct((1, 1), jnp.float32),
    )(pcat, jnp.asarray(_CLS_PLANE), z.astype(jnp.bfloat16))

    return loss_cls[0, 0] + WL * reg[0, 0]


# speedup vs baseline: 1.6665x; 1.0728x over previous
import math

import jax
import jax.numpy as jnp
import numpy as np
from jax.experimental import pallas as pl
from jax.experimental.pallas import tpu as pltpu

B = 1024
DIM = 512
C = 1211
K = 3
CN = C * K
R = 0.4
WL = 0.3
TOPK = math.ceil(R * CN)

Cp = 1280
W = K * Cp
RB = 128
NB = B // RB
CT = 256
NT = W // CT

_CLS_PLANE = (np.arange(W, dtype=np.int32) % Cp).reshape(1, W)
_CLS_CP = np.arange(Cp, dtype=np.int32).reshape(1, Cp)


def _fused_kernel(x_ref, p_ref, tgt_ref, ca_ref, cc_ref, out_ref, z_ref,
                  inv_ref):
    pid = pl.program_id(0)

    @pl.when(pid == 0)
    def _():
        pf = p_ref[...].astype(jnp.float32)
        ssq = jnp.sum(pf * pf, axis=0, keepdims=True)
        inv = 1.0 / jnp.maximum(jnp.sqrt(ssq), 1e-12)
        inv_ref[...] = inv
        t = pf * inv
        z_ref[...] = (t[:, :Cp] + t[:, Cp:2 * Cp]
                      + t[:, 2 * Cp:]).astype(jnp.bfloat16)
        out_ref[...] = jnp.zeros((1, 1), jnp.float32)

    @pl.when(pid < NB)
    def _loss_phase():
        x = x_ref[...]
        s = jax.lax.dot_general(x, p_ref[...], (((1,), (0,)), ((), ())),
                                preferred_element_type=jnp.float32)
        s = s * inv_ref[...]

        cls_all = ca_ref[...]
        cls_cp = cc_ref[...]
        tgt = tgt_ref[:, 0:1]

        boosted = jnp.where(cls_all < C,
                            s + jnp.where(cls_all == tgt, 100.0, 0.0),
                            -3e38)
        u = jax.lax.bitcast_convert_type(boosted, jnp.int32)
        es = jnp.where(u >= 0, u, u ^ np.int32(0x7FFFFFFF))

        topk16 = np.int16(TOPK)
        bias16 = np.uint16(0x8000).view(np.int16)

        def count16(mask):
            ones = jnp.where(mask, jnp.bfloat16(1), jnp.bfloat16(0))
            acc = ones[:, :128]
            for i in range(1, W // 128):
                acc = acc + ones[:, i * 128:(i + 1) * 128]
            cnt = jnp.sum(acc.astype(jnp.float32), axis=1, keepdims=True)
            return cnt.astype(jnp.int32).astype(jnp.int16)

        hi = (es >> 16).astype(jnp.int16)
        t_hi_u = jnp.zeros((RB, 1), jnp.int16)
        for b in range(15, -1, -1):
            cand_u = t_hi_u | np.uint16(1 << b).view(np.int16)
            cnt = count16(hi >= (cand_u ^ bias16))
            t_hi_u = jnp.where(cnt >= topk16, cand_u, t_hi_u)
        t_hi = t_hi_u ^ bias16

        c_eq = count16(hi == t_hi)
        c_ge = count16(hi >= t_hi)
        kk2 = topk16 - (c_ge - c_eq)

        lo16 = ((es & np.int32(0xFFFF)) - 32768).astype(jnp.int16)
        act = jnp.where(hi == t_hi, lo16, np.int16(-32768))
        t_lo_u = jnp.zeros((RB, 1), jnp.int16)
        for b in range(15, -1, -1):
            cand_u = t_lo_u | np.uint16(1 << b).view(np.int16)
            cnt = count16(act >= (cand_u ^ bias16))
            t_lo_u = jnp.where(cnt >= kk2, cand_u, t_lo_u)

        t_s = ((t_hi.astype(jnp.int32)) << 16) \
            | (t_lo_u.astype(jnp.int32) & 0xFFFF)

        m = jnp.where(es >= t_s, s, 0.0)
        logits = m[:, :Cp] + m[:, Cp:2 * Cp] + m[:, 2 * Cp:]

        se = jnp.where(logits != 0.0, jnp.exp(logits), 0.0)
        denom = 1e-8 + jnp.sum(se, axis=1, keepdims=True)
        texp = jnp.sum(jnp.where(cls_cp == tgt, se, 0.0),
                       axis=1, keepdims=True)
        lossrow = -jnp.log(texp / denom + 1e-20)
        out_ref[...] += (jnp.sum(lossrow, keepdims=True).reshape(1, 1)
                         * (1.0 / B))

    @pl.when(pid >= NB)
    def _reg_phase():
        j = (pid - NB) * CT
        pb = p_ref[:, pl.ds(j, CT)]
        z16 = z_ref[...]
        inv_t = inv_ref[:, pl.ds(j, CT)]
        ci = ca_ref[:, pl.ds(j, CT)]

        clt = jax.lax.dot_general(z16, pb, (((0,), (0,)), ((), ())),
                                  preferred_element_type=jnp.float32)
        clt = clt * inv_t

        rows = jax.lax.broadcasted_iota(jnp.int32, (Cp, CT), 0)
        clm = jnp.where(rows < C, clt, -3e38)
        mx = jnp.max(clm, axis=0, keepdims=True)
        lse = mx + jnp.log(jnp.sum(jnp.exp(clm - mx), axis=0, keepdims=True))
        diag = jnp.sum(jnp.where(rows == ci, clt, 0.0), axis=0, keepdims=True)
        contrib = jnp.where(ci < C, lse - diag, 0.0)

        out_ref[...] += (jnp.sum(contrib, keepdims=True).reshape(1, 1)
                         * (WL / CN))


@jax.jit
def kernel(input, proxies, target):
    pr = proxies.reshape(DIM, C, K)
    planes = [jnp.pad(pr[:, :, k], ((0, 0), (0, Cp - C))) for k in range(K)]
    pcat = jnp.concatenate(planes, axis=1).astype(jnp.bfloat16)
    x16 = input.astype(jnp.bfloat16)
    tgt2d = jnp.broadcast_to(target[:, None], (B, RB)).astype(jnp.int32)

    out, _, _ = pl.pallas_call(
        _fused_kernel,
        grid=(NB + NT,),
        in_specs=[
            pl.BlockSpec((RB, DIM), lambda i: (jnp.minimum(i, NB - 1), 0)),
            pl.BlockSpec((DIM, W), lambda i: (0, 0)),
            pl.BlockSpec((RB, RB), lambda i: (jnp.minimum(i, NB - 1), 0)),
            pl.BlockSpec((1, W), lambda i: (0, 0)),
            pl.BlockSpec((1, Cp), lambda i: (0, 0)),
        ],
        out_specs=(
            pl.BlockSpec((1, 1), lambda i: (0, 0)),
            pl.BlockSpec((DIM, Cp), lambda i: (0, 0)),
            pl.BlockSpec((1, W), lambda i: (0, 0)),
        ),
        out_shape=(
            jax.ShapeDtypeStruct((1, 1), jnp.float32),
            jax.ShapeDtypeStruct((DIM, Cp), jnp.bfloat16),
            jax.ShapeDtypeStruct((1, W), jnp.float32),
        ),
    )(x16, pcat, tgt2d, jnp.asarray(_CLS_PLANE), jnp.asarray(_CLS_CP))

    return out[0, 0]


# speedup vs baseline: 1.7519x; 1.0513x over previous
import math

import jax
import jax.numpy as jnp
import numpy as np
from jax.experimental import pallas as pl
from jax.experimental.pallas import tpu as pltpu

B = 1024
DIM = 512
C = 1211
K = 3
CN = C * K
R = 0.4
WL = 0.3
TOPK = math.ceil(R * CN)

Cp = 1280
W = K * Cp
RB = 256
NB = B // RB
CT = 256
NT = W // CT

_CLS_PLANE = (np.arange(W, dtype=np.int32) % Cp).reshape(1, W)
_CLS_CP = np.arange(Cp, dtype=np.int32).reshape(1, Cp)


def _fused_kernel(x_ref, p_ref, tgt_ref, ca_ref, cc_ref, out_ref, z_ref,
                  inv_ref):
    pid = pl.program_id(0)

    @pl.when(pid == 0)
    def _():
        pf = p_ref[...].astype(jnp.float32)
        ssq = jnp.sum(pf * pf, axis=0, keepdims=True)
        inv = 1.0 / jnp.maximum(jnp.sqrt(ssq), 1e-12)
        inv_ref[...] = inv
        t = pf * inv
        z_ref[...] = (t[:, :Cp] + t[:, Cp:2 * Cp]
                      + t[:, 2 * Cp:]).astype(jnp.bfloat16)
        out_ref[...] = jnp.zeros((1, 1), jnp.float32)

    @pl.when(pid < NB)
    def _loss_phase():
        x = x_ref[...]
        s = jax.lax.dot_general(x, p_ref[...], (((1,), (0,)), ((), ())),
                                preferred_element_type=jnp.float32)
        s = s * inv_ref[...]

        cls_all = ca_ref[...]
        cls_cp = cc_ref[...]
        tgt = tgt_ref[:, 0:1]

        boosted = jnp.where(cls_all < C,
                            s + jnp.where(cls_all == tgt, 100.0, 0.0),
                            -3e38)
        u = jax.lax.bitcast_convert_type(boosted, jnp.int32)
        es = jnp.where(u >= 0, u, u ^ np.int32(0x7FFFFFFF))

        topk16 = np.int16(TOPK)
        bias16 = np.uint16(0x8000).view(np.int16)

        def count16(mask):
            ones = jnp.where(mask, jnp.bfloat16(1), jnp.bfloat16(0))
            acc = ones[:, :128]
            for i in range(1, W // 128):
                acc = acc + ones[:, i * 128:(i + 1) * 128]
            cnt = jnp.sum(acc.astype(jnp.float32), axis=1, keepdims=True)
            return cnt.astype(jnp.int32).astype(jnp.int16)

        hi = (es >> 16).astype(jnp.int16)
        t_hi_u = jnp.zeros((RB, 1), jnp.int16)
        for b in range(15, -1, -1):
            cand_u = t_hi_u | np.uint16(1 << b).view(np.int16)
            cnt = count16(hi >= (cand_u ^ bias16))
            t_hi_u = jnp.where(cnt >= topk16, cand_u, t_hi_u)
        t_hi = t_hi_u ^ bias16

        c_eq = count16(hi == t_hi)
        c_ge = count16(hi >= t_hi)
        kk2 = topk16 - (c_ge - c_eq)

        lo16 = ((es & np.int32(0xFFFF)) - 32768).astype(jnp.int16)
        act = jnp.where(hi == t_hi, lo16, np.int16(-32768))
        t_lo_u = jnp.zeros((RB, 1), jnp.int16)
        for b in range(15, -1, -1):
            cand_u = t_lo_u | np.uint16(1 << b).view(np.int16)
            cnt = count16(act >= (cand_u ^ bias16))
            t_lo_u = jnp.where(cnt >= kk2, cand_u, t_lo_u)

        t_s = ((t_hi.astype(jnp.int32)) << 16) \
            | (t_lo_u.astype(jnp.int32) & 0xFFFF)

        m = jnp.where(es >= t_s, s, 0.0)
        logits = m[:, :Cp] + m[:, Cp:2 * Cp] + m[:, 2 * Cp:]

        se = jnp.where(logits != 0.0, jnp.exp(logits), 0.0)
        denom = 1e-8 + jnp.sum(se, axis=1, keepdims=True)
        texp = jnp.sum(jnp.where(cls_cp == tgt, se, 0.0),
                       axis=1, keepdims=True)
        lossrow = -jnp.log(texp / denom + 1e-20)
        out_ref[...] += (jnp.sum(lossrow, keepdims=True).reshape(1, 1)
                         * (1.0 / B))

    @pl.when(pid >= NB)
    def _reg_phase():
        j = (pid - NB) * CT
        pb = p_ref[:, pl.ds(j, CT)]
        z16 = z_ref[...]
        inv_t = inv_ref[:, pl.ds(j, CT)]
        ci = ca_ref[:, pl.ds(j, CT)]

        clt = jax.lax.dot_general(z16, pb, (((0,), (0,)), ((), ())),
                                  preferred_element_type=jnp.float32)
        clt = clt * inv_t

        rows = jax.lax.broadcasted_iota(jnp.int32, (Cp, CT), 0)
        clm = jnp.where(rows < C, clt, -3e38)
        mx = jnp.max(clm, axis=0, keepdims=True)
        lse = mx + jnp.log(jnp.sum(jnp.exp(clm - mx), axis=0, keepdims=True))
        diag = jnp.sum(jnp.where(rows == ci, clt, 0.0), axis=0, keepdims=True)
        contrib = jnp.where(ci < C, lse - diag, 0.0)

        out_ref[...] += (jnp.sum(contrib, keepdims=True).reshape(1, 1)
                         * (WL / CN))


@jax.jit
def kernel(input, proxies, target):
    pr = proxies.astype(jnp.bfloat16).reshape(DIM, C, K)
    planes = [jnp.pad(pr[:, :, k], ((0, 0), (0, Cp - C))) for k in range(K)]
    pcat = jnp.concatenate(planes, axis=1)
    x16 = input.astype(jnp.bfloat16)
    tgt2d = jnp.broadcast_to(target[:, None], (B, RB)).astype(jnp.int32)

    out, _, _ = pl.pallas_call(
        _fused_kernel,
        grid=(NB + NT,),
        in_specs=[
            pl.BlockSpec((RB, DIM), lambda i: (jnp.minimum(i, NB - 1), 0)),
            pl.BlockSpec((DIM, W), lambda i: (0, 0)),
            pl.BlockSpec((RB, RB), lambda i: (jnp.minimum(i, NB - 1), 0)),
            pl.BlockSpec((1, W), lambda i: (0, 0)),
            pl.BlockSpec((1, Cp), lambda i: (0, 0)),
        ],
        out_specs=(
            pl.BlockSpec((1, 1), lambda i: (0, 0)),
            pl.BlockSpec((DIM, Cp), lambda i: (0, 0)),
            pl.BlockSpec((1, W), lambda i: (0, 0)),
        ),
        out_shape=(
            jax.ShapeDtypeStruct((1, 1), jnp.float32),
            jax.ShapeDtypeStruct((DIM, Cp), jnp.bfloat16),
            jax.ShapeDtypeStruct((1, W), jnp.float32),
        ),
    )(x16, pcat, tgt2d, jnp.asarray(_CLS_PLANE), jnp.asarray(_CLS_CP))

    return out[0, 0]


# speedup vs baseline: 1.7765x; 1.0140x over previous
import math

import jax
import jax.numpy as jnp
import numpy as np
from jax.experimental import pallas as pl
from jax.experimental.pallas import tpu as pltpu

B = 1024
DIM = 512
C = 1211
K = 3
CN = C * K
R = 0.4
WL = 0.3
TOPK = math.ceil(R * CN)

Cp = 1280
W = K * Cp
RB = 512
NB = B // RB
CT = 384
NT = W // CT

_CLS_PLANE = (np.arange(W, dtype=np.int32) % Cp).reshape(1, W)
_CLS_CP = np.arange(Cp, dtype=np.int32).reshape(1, Cp)


def _fused_kernel(x_ref, p_ref, tgt_ref, ca_ref, cc_ref, out_ref, z_ref,
                  inv_ref):
    pid = pl.program_id(0)

    @pl.when(pid == 0)
    def _():
        pf = p_ref[...].astype(jnp.float32)
        ssq = jnp.sum(pf * pf, axis=0, keepdims=True)
        inv = 1.0 / jnp.maximum(jnp.sqrt(ssq), 1e-12)
        inv_ref[...] = inv
        t = pf * inv
        z_ref[...] = (t[:, :Cp] + t[:, Cp:2 * Cp]
                      + t[:, 2 * Cp:]).astype(jnp.bfloat16)
        out_ref[...] = jnp.zeros((1, 1), jnp.float32)

    @pl.when(pid < NB)
    def _loss_phase():
        x = x_ref[...]
        s = jax.lax.dot_general(x, p_ref[...], (((1,), (0,)), ((), ())),
                                preferred_element_type=jnp.float32)
        s = s * inv_ref[...]

        cls_all = ca_ref[...]
        cls_cp = cc_ref[...]
        tgt = tgt_ref[:, 0:1]

        boosted = jnp.where(cls_all < C,
                            s + jnp.where(cls_all == tgt, 100.0, 0.0),
                            -3e38)
        u = jax.lax.bitcast_convert_type(boosted, jnp.int32)
        es = jnp.where(u >= 0, u, u ^ np.int32(0x7FFFFFFF))

        topk16 = np.int16(TOPK)
        bias16 = np.uint16(0x8000).view(np.int16)

        def count16(mask):
            ones = jnp.where(mask, jnp.bfloat16(1), jnp.bfloat16(0))
            acc = ones[:, :128]
            for i in range(1, W // 128):
                acc = acc + ones[:, i * 128:(i + 1) * 128]
            cnt = jnp.sum(acc.astype(jnp.float32), axis=1, keepdims=True)
            return cnt.astype(jnp.int32).astype(jnp.int16)

        hi = (es >> 16).astype(jnp.int16)
        t_hi_u = jnp.zeros((RB, 1), jnp.int16)
        for b in range(15, -1, -1):
            cand_u = t_hi_u | np.uint16(1 << b).view(np.int16)
            cnt = count16(hi >= (cand_u ^ bias16))
            t_hi_u = jnp.where(cnt >= topk16, cand_u, t_hi_u)
        t_hi = t_hi_u ^ bias16

        c_eq = count16(hi == t_hi)
        c_ge = count16(hi >= t_hi)
        kk2 = topk16 - (c_ge - c_eq)

        lo16 = ((es & np.int32(0xFFFF)) - 32768).astype(jnp.int16)
        act = jnp.where(hi == t_hi, lo16, np.int16(-32768))
        t_lo_u = jnp.zeros((RB, 1), jnp.int16)
        for b in range(15, -1, -1):
            cand_u = t_lo_u | np.uint16(1 << b).view(np.int16)
            cnt = count16(act >= (cand_u ^ bias16))
            t_lo_u = jnp.where(cnt >= kk2, cand_u, t_lo_u)

        t_s = ((t_hi.astype(jnp.int32)) << 16) \
            | (t_lo_u.astype(jnp.int32) & 0xFFFF)

        m = jnp.where(es >= t_s, s, 0.0)
        logits = m[:, :Cp] + m[:, Cp:2 * Cp] + m[:, 2 * Cp:]

        se = jnp.where(logits != 0.0, jnp.exp(logits), 0.0)
        denom = 1e-8 + jnp.sum(se, axis=1, keepdims=True)
        texp = jnp.sum(jnp.where(cls_cp == tgt, se, 0.0),
                       axis=1, keepdims=True)
        lossrow = -jnp.log(texp / denom + 1e-20)
        out_ref[...] += (jnp.sum(lossrow, keepdims=True).reshape(1, 1)
                         * (1.0 / B))

    @pl.when(pid >= NB)
    def _reg_phase():
        j = (pid - NB) * CT
        pb = p_ref[:, pl.ds(j, CT)]
        z16 = z_ref[...]
        inv_t = inv_ref[:, pl.ds(j, CT)]
        ci = ca_ref[:, pl.ds(j, CT)]

        clt = jax.lax.dot_general(z16, pb, (((0,), (0,)), ((), ())),
                                  preferred_element_type=jnp.float32)
        clt = clt * inv_t

        rows = jax.lax.broadcasted_iota(jnp.int32, (Cp, CT), 0)
        clm = jnp.where(rows < C, clt, -3e38)
        mx = jnp.max(clm, axis=0, keepdims=True)
        lse = mx + jnp.log(jnp.sum(jnp.exp(clm - mx), axis=0, keepdims=True))
        diag = jnp.sum(jnp.where(rows == ci, clt, 0.0), axis=0, keepdims=True)
        contrib = jnp.where(ci < C, lse - diag, 0.0)

        out_ref[...] += (jnp.sum(contrib, keepdims=True).reshape(1, 1)
                         * (WL / CN))


@jax.jit
def kernel(input, proxies, target):
    pr = proxies.astype(jnp.bfloat16).reshape(DIM, C, K)
    planes = [jnp.pad(pr[:, :, k], ((0, 0), (0, Cp - C))) for k in range(K)]
    pcat = jnp.concatenate(planes, axis=1)
    x16 = input.astype(jnp.bfloat16)
    tgt2d = jnp.broadcast_to(target[:, None], (B, RB)).astype(jnp.int32)

    out, _, _ = pl.pallas_call(
        _fused_kernel,
        grid=(NB + NT,),
        in_specs=[
            pl.BlockSpec((RB, DIM), lambda i: (jnp.minimum(i, NB - 1), 0)),
            pl.BlockSpec((DIM, W), lambda i: (0, 0)),
            pl.BlockSpec((RB, RB), lambda i: (jnp.minimum(i, NB - 1), 0)),
            pl.BlockSpec((1, W), lambda i: (0, 0)),
            pl.BlockSpec((1, Cp), lambda i: (0, 0)),
        ],
        out_specs=(
            pl.BlockSpec((1, 1), lambda i: (0, 0)),
            pl.BlockSpec((DIM, Cp), lambda i: (0, 0)),
            pl.BlockSpec((1, W), lambda i: (0, 0)),
        ),
        out_shape=(
            jax.ShapeDtypeStruct((1, 1), jnp.float32),
            jax.ShapeDtypeStruct((DIM, Cp), jnp.bfloat16),
            jax.ShapeDtypeStruct((1, W), jnp.float32),
        ),
    )(x16, pcat, tgt2d, jnp.asarray(_CLS_PLANE), jnp.asarray(_CLS_CP))

    return out[0, 0]


# speedup vs baseline: 1.7774x; 1.0005x over previous
import math

import jax
import jax.numpy as jnp
import numpy as np
from jax.experimental import pallas as pl

B = 1024
DIM = 512
C = 1211
K = 3
CN = C * K
R = 0.4
WL = 0.3
TOPK = math.ceil(R * CN)

Cp = 1280
W = K * Cp
RB = 512
NB = B // RB
CT = 384
NT = W // CT

_CLS_PLANE = (np.arange(W, dtype=np.int32) % Cp).reshape(1, W)
_CLS_CP = np.arange(Cp, dtype=np.int32).reshape(1, Cp)


def _fused_kernel(x_ref, p_ref, tgt_ref, ca_ref, cc_ref, out_ref, z_ref,
                  inv_ref):
    pid = pl.program_id(0)

    @pl.when(pid == 0)
    def _():
        pf = p_ref[...].astype(jnp.float32)
        ssq = jnp.sum(pf * pf, axis=0, keepdims=True)
        inv = 1.0 / jnp.maximum(jnp.sqrt(ssq), 1e-12)
        inv_ref[...] = inv
        t = pf * inv
        z_ref[...] = (t[:, :Cp] + t[:, Cp:2 * Cp]
                      + t[:, 2 * Cp:]).astype(jnp.bfloat16)
        out_ref[...] = jnp.zeros((1, 1), jnp.float32)

    @pl.when(pid < NB)
    def _loss_phase():
        x = x_ref[...]
        s = jax.lax.dot_general(x, p_ref[...], (((1,), (0,)), ((), ())),
                                preferred_element_type=jnp.float32)
        s = s * inv_ref[...]

        cls_all = ca_ref[...]
        cls_cp = cc_ref[...]
        tgt = tgt_ref[:, 0:1]

        boosted = jnp.where(cls_all < C,
                            s + jnp.where(cls_all == tgt, 100.0, 0.0),
                            -3e38)
        u = jax.lax.bitcast_convert_type(boosted, jnp.int32)
        es = jnp.where(u >= 0, u, u ^ np.int32(0x7FFFFFFF))

        topk16 = np.int16(TOPK)
        bias16 = np.uint16(0x8000).view(np.int16)

        def count16(mask):
            ones = jnp.where(mask, jnp.bfloat16(1), jnp.bfloat16(0))
            acc = ones[:, :128]
            for i in range(1, W // 128):
                acc = acc + ones[:, i * 128:(i + 1) * 128]
            cnt = jnp.sum(acc.astype(jnp.float32), axis=1, keepdims=True)
            return cnt.astype(jnp.int32).astype(jnp.int16)

        hi = (es >> 16).astype(jnp.int16)
        t_hi_u = jnp.zeros((RB, 1), jnp.int16)
        for b in range(15, -1, -1):
            cand_u = t_hi_u | np.uint16(1 << b).view(np.int16)
            cnt = count16(hi >= (cand_u ^ bias16))
            t_hi_u = jnp.where(cnt >= topk16, cand_u, t_hi_u)
        t_hi = t_hi_u ^ bias16

        c_eq = count16(hi == t_hi)
        c_ge = count16(hi >= t_hi)
        kk2 = topk16 - (c_ge - c_eq)

        lo16 = ((es & np.int32(0xFFFF)) - 32768).astype(jnp.int16)
        act = jnp.where(hi == t_hi, lo16, np.int16(-32768))
        t_lo_u = jnp.zeros((RB, 1), jnp.int16)
        for b in range(15, -1, -1):
            cand_u = t_lo_u | np.uint16(1 << b).view(np.int16)
            cnt = count16(act >= (cand_u ^ bias16))
            t_lo_u = jnp.where(cnt >= kk2, cand_u, t_lo_u)

        t_s = ((t_hi.astype(jnp.int32)) << 16) \
            | (t_lo_u.astype(jnp.int32) & 0xFFFF)

        m = jnp.where(es >= t_s, s, 0.0)
        logits = m[:, :Cp] + m[:, Cp:2 * Cp] + m[:, 2 * Cp:]

        se = jnp.where(logits != 0.0, jnp.exp(logits), 0.0)
        denom = 1e-8 + jnp.sum(se, axis=1, keepdims=True)
        texp = jnp.sum(jnp.where(cls_cp == tgt, se, 0.0),
                       axis=1, keepdims=True)
        lossrow = -jnp.log(texp / denom + 1e-20)
        out_ref[...] += (jnp.sum(lossrow, keepdims=True).reshape(1, 1)
                         * (1.0 / B))

    @pl.when(pid >= NB)
    def _reg_phase():
        j = (pid - NB) * CT
        pb = p_ref[:, pl.ds(j, CT)]
        z16 = z_ref[...]
        inv_t = inv_ref[:, pl.ds(j, CT)]
        ci = ca_ref[:, pl.ds(j, CT)]

        clt = jax.lax.dot_general(z16, pb, (((0,), (0,)), ((), ())),
                                  preferred_element_type=jnp.float32)
        clt = clt * inv_t

        rows = jax.lax.broadcasted_iota(jnp.int32, (Cp, CT), 0)
        clm = jnp.where(rows < C, clt, -3e38)
        mx = jnp.max(clm, axis=0, keepdims=True)
        lse = mx + jnp.log(jnp.sum(jnp.exp(clm - mx), axis=0, keepdims=True))
        diag = jnp.sum(jnp.where(rows == ci, clt, 0.0), axis=0, keepdims=True)
        contrib = jnp.where(ci < C, lse - diag, 0.0)

        out_ref[...] += (jnp.sum(contrib, keepdims=True).reshape(1, 1)
                         * (WL / CN))


@jax.jit
def kernel(input, proxies, target):
    pr = proxies.astype(jnp.bfloat16).reshape(DIM, C, K)
    planes = [jnp.pad(pr[:, :, k], ((0, 0), (0, Cp - C))) for k in range(K)]
    pcat = jnp.concatenate(planes, axis=1)
    x16 = input.astype(jnp.bfloat16)
    tgt2d = jnp.broadcast_to(target[:, None], (B, RB)).astype(jnp.int32)

    out, _, _ = pl.pallas_call(
        _fused_kernel,
        grid=(NB + NT,),
        in_specs=[
            pl.BlockSpec((RB, DIM), lambda i: (jnp.minimum(i, NB - 1), 0)),
            pl.BlockSpec((DIM, W), lambda i: (0, 0)),
            pl.BlockSpec((RB, RB), lambda i: (jnp.minimum(i, NB - 1), 0)),
            pl.BlockSpec((1, W), lambda i: (0, 0)),
            pl.BlockSpec((1, Cp), lambda i: (0, 0)),
        ],
        out_specs=(
            pl.BlockSpec((1, 1), lambda i: (0, 0)),
            pl.BlockSpec((DIM, Cp), lambda i: (0, 0)),
            pl.BlockSpec((1, W), lambda i: (0, 0)),
        ),
        out_shape=(
            jax.ShapeDtypeStruct((1, 1), jnp.float32),
            jax.ShapeDtypeStruct((DIM, Cp), jnp.bfloat16),
            jax.ShapeDtypeStruct((1, W), jnp.float32),
        ),
    )(x16, pcat, tgt2d, jnp.asarray(_CLS_PLANE), jnp.asarray(_CLS_CP))

    return out[0, 0]
